# baseline (speedup 1.0000x reference)
import functools

import jax
import jax.numpy as jnp
from jax.experimental import pallas as pl
from jax.experimental.pallas import tpu as pltpu

_K = 8
_NEG = -1e30


def _fused_kernel(x_ref, gain_ref, bias_ref, lm_ref, ls_ref, out_ref, xn_ref,
                  w_ref, msg_ref, m_ref, acc_ref, deg_ref, *, blk, nchunks):
    i = pl.program_id(1)

    @pl.when(i == 0)
    def _normalize():
        xa_full = x_ref[0]
        n2 = jnp.sum(xa_full * xa_full, axis=1, keepdims=True)
        xn_ref[...] = xa_full / (jnp.sqrt(n2) + 1e-8)

    row0 = i * blk
    xn_rows = xn_ref[pl.ds(row0, blk), :]

    for c in range(nchunks):
        @pl.when(c <= i)
        def _build(c=c):
            simc = jax.lax.dot_general(
                xn_rows, xn_ref[c * blk:(c + 1) * blk, :],
                (((1,), (1,)), ((), ())),
                preferred_element_type=jnp.float32)
            cols = c * blk + jax.lax.broadcasted_iota(jnp.int32, (blk, blk), 1)
            rows = row0 + jax.lax.broadcasted_iota(jnp.int32, (blk, blk), 0)
            w_ref[:, c * blk:(c + 1) * blk] = jnp.where(cols <= rows, simc,
                                                        _NEG)

    m_ref[...] = jnp.full_like(m_ref, 1e30)
    for _ in range(_K):
        acc_ref[...] = jnp.full_like(acc_ref, _NEG)
        for c in range(nchunks):
            @pl.when(c <= i)
            def _round(c=c):
                wc = w_ref[:, c * blk:(c + 1) * blk]
                t = jnp.where(wc < m_ref[...], wc, _NEG)
                acc_ref[...] = jnp.maximum(
                    acc_ref[...], jnp.max(t, axis=1, keepdims=True))
        m_ref[...] = acc_ref[...]
    thresh = jnp.maximum(m_ref[...], -2.0)

    msg_ref[...] = jnp.zeros_like(msg_ref)
    deg_ref[...] = jnp.zeros_like(deg_ref)
    for c in range(nchunks):
        @pl.when(c <= i)
        def _agg(c=c):
            wc = w_ref[:, c * blk:(c + 1) * blk]
            adjc = jnp.where(wc >= thresh, 1.0, 0.0)
            msg_ref[...] += jax.lax.dot_general(
                adjc, x_ref[0, c * blk:(c + 1) * blk, :],
                (((1,), (0,)), ((), ())),
                preferred_element_type=jnp.float32)
            deg_ref[...] += jnp.sum(adjc, axis=1, keepdims=True)

    msg = msg_ref[...] / jnp.maximum(deg_ref[...], 1.0)

    mix = jax.nn.sigmoid(lm_ref[0, 0])
    scale = jax.nn.softplus(ls_ref[0, 0]) + 0.01

    x_rows = x_ref[0, pl.ds(row0, blk), :]
    blended = mix * x_rows + (1.0 - mix) * msg
    y = blended * gain_ref[0][None, :] + bias_ref[0][None, :]
    gelu = 0.5 * y * (1.0 + jax.lax.erf(y * (2.0 ** -0.5)))
    out_ref[0] = gelu * scale


def kernel(x, gain, bias, log_mix, log_scale):
    B, T, D = x.shape
    blk = min(256, T)
    grid = (B, T // blk)

    fn = functools.partial(_fused_kernel, blk=blk, nchunks=T // blk)
    return pl.pallas_call(
        fn,
        grid=grid,
        in_specs=[
            pl.BlockSpec((1, T, D), lambda b, i: (b, 0, 0)),
            pl.BlockSpec((1, D), lambda b, i: (0, 0)),
            pl.BlockSpec((1, D), lambda b, i: (0, 0)),
            pl.BlockSpec((1, 1), lambda b, i: (0, 0)),
            pl.BlockSpec((1, 1), lambda b, i: (0, 0)),
        ],
        out_specs=pl.BlockSpec((1, blk, D), lambda b, i: (b, i, 0)),
        out_shape=jax.ShapeDtypeStruct((B, T, D), x.dtype),
        scratch_shapes=[
            pltpu.VMEM((T, D), jnp.float32),
            pltpu.VMEM((blk, T), jnp.float32),
            pltpu.VMEM((blk, D), jnp.float32),
            pltpu.VMEM((blk, 1), jnp.float32),
            pltpu.VMEM((blk, 1), jnp.float32),
            pltpu.VMEM((blk, 1), jnp.float32),
        ],
    )(x, gain.reshape(1, D), bias.reshape(1, D),
      log_mix.reshape(1, 1), log_scale.reshape(1, 1))


# speedup vs baseline: 2.9265x; 2.9265x over previous
import functools

import jax
import jax.numpy as jnp
from jax.experimental import pallas as pl
from jax.experimental.pallas import tpu as pltpu

_K = 8
_NEG = -1e30
_NPATH = 4


def _fused_kernel(x_ref, gain_ref, bias_ref, lm_ref, ls_ref, out_ref, xn_ref,
                  *, blk):
    i = pl.program_id(1)

    @pl.when(i == 0)
    def _normalize():
        xa_full = x_ref[0]
        n2 = jnp.sum(xa_full * xa_full, axis=1, keepdims=True)
        xn_ref[...] = xa_full / (jnp.sqrt(n2) + 1e-8)

    row0 = i * blk
    mix = jax.nn.sigmoid(lm_ref[0, 0])
    scale = jax.nn.softplus(ls_ref[0, 0]) + 0.01

    def _path(wcols):
        xn_rows = xn_ref[pl.ds(row0, blk), :]
        sim = jax.lax.dot_general(
            xn_rows, xn_ref[:wcols, :], (((1,), (1,)), ((), ())),
            preferred_element_type=jnp.float32)

        cols = jax.lax.broadcasted_iota(jnp.int32, (blk, wcols), 1)
        rows = row0 + jax.lax.broadcasted_iota(jnp.int32, (blk, wcols), 0)
        w = jnp.where(cols <= rows, sim, _NEG)

        m = jnp.max(w, axis=1, keepdims=True)
        for _ in range(_K - 1):
            m = jnp.max(jnp.where(w < m, w, _NEG), axis=1, keepdims=True)
        thresh = jnp.maximum(m, -2.0)

        adj = jnp.where(w >= thresh, 1.0, 0.0)
        deg = jnp.sum(adj, axis=1, keepdims=True)

        msg = jax.lax.dot_general(
            adj, x_ref[0, :wcols, :], (((1,), (0,)), ((), ())),
            preferred_element_type=jnp.float32)
        msg = msg / jnp.maximum(deg, 1.0)

        x_rows = x_ref[0, pl.ds(row0, blk), :]
        blended = mix * x_rows + (1.0 - mix) * msg
        y = blended * gain_ref[0][None, :] + bias_ref[0][None, :]
        gelu = 0.5 * y * (1.0 + jax.lax.erf(y * (2.0 ** -0.5)))
        out_ref[0] = gelu * scale

    t_total = x_ref.shape[1]
    nblk = t_total // blk
    per_path = nblk // _NPATH
    for p in range(_NPATH):
        lo, hi = p * per_path, (p + 1) * per_path
        cond = (i >= lo) & (i < hi) if p else (i < hi)

        @pl.when(cond)
        def _run(p=p):
            _path((p + 1) * per_path * blk)


def kernel(x, gain, bias, log_mix, log_scale):
    B, T, D = x.shape
    blk = min(256, T)
    grid = (B, T // blk)

    fn = functools.partial(_fused_kernel, blk=blk)
    return pl.pallas_call(
        fn,
        grid=grid,
        in_specs=[
            pl.BlockSpec((1, T, D), lambda b, i: (b, 0, 0)),
            pl.BlockSpec((1, D), lambda b, i: (0, 0)),
            pl.BlockSpec((1, D), lambda b, i: (0, 0)),
            pl.BlockSpec((1, 1), lambda b, i: (0, 0)),
            pl.BlockSpec((1, 1), lambda b, i: (0, 0)),
        ],
        out_specs=pl.BlockSpec((1, blk, D), lambda b, i: (b, i, 0)),
        out_shape=jax.ShapeDtypeStruct((B, T, D), x.dtype),
        scratch_shapes=[pltpu.VMEM((T, D), jnp.float32)],
    )(x, gain.reshape(1, D), bias.reshape(1, D),
      log_mix.reshape(1, 1), log_scale.reshape(1, 1))


# speedup vs baseline: 3.0846x; 1.0540x over previous
import functools

import jax
import jax.numpy as jnp
from jax.experimental import pallas as pl
from jax.experimental.pallas import tpu as pltpu

_K = 8
_NEG = -1e30
_NPATH = 8


def _fused_kernel(x_ref, gain_ref, bias_ref, lm_ref, ls_ref, out_ref, xn_ref,
                  *, blk):
    i = pl.program_id(1)

    @pl.when(i == 0)
    def _normalize():
        xa_full = x_ref[0]
        n2 = jnp.sum(xa_full * xa_full, axis=1, keepdims=True)
        xn_ref[...] = xa_full / (jnp.sqrt(n2) + 1e-8)

    row0 = i * blk
    mix = jax.nn.sigmoid(lm_ref[0, 0])
    scale = jax.nn.softplus(ls_ref[0, 0]) + 0.01

    def _path(wcols):
        xn_rows = xn_ref[pl.ds(row0, blk), :]
        sim = jax.lax.dot_general(
            xn_rows, xn_ref[:wcols, :], (((1,), (1,)), ((), ())),
            preferred_element_type=jnp.float32)

        cols = jax.lax.broadcasted_iota(jnp.int32, (blk, wcols), 1)
        rows = row0 + jax.lax.broadcasted_iota(jnp.int32, (blk, wcols), 0)
        w = jnp.where(cols <= rows, sim, _NEG)

        m = jnp.max(w, axis=1, keepdims=True)
        for _ in range(_K - 1):
            m = jnp.max(jnp.where(w < m, w, _NEG), axis=1, keepdims=True)
        thresh = jnp.maximum(m, -2.0)

        adj = jnp.where(w >= thresh, 1.0, 0.0)
        deg = jnp.sum(adj, axis=1, keepdims=True)

        msg = jax.lax.dot_general(
            adj, x_ref[0, :wcols, :], (((1,), (0,)), ((), ())),
            preferred_element_type=jnp.float32)
        msg = msg / jnp.maximum(deg, 1.0)

        x_rows = x_ref[0, pl.ds(row0, blk), :]
        blended = mix * x_rows + (1.0 - mix) * msg
        y = blended * gain_ref[0][None, :] + bias_ref[0][None, :]
        gelu = 0.5 * y * (1.0 + jax.lax.erf(y * (2.0 ** -0.5)))
        out_ref[0] = gelu * scale

    t_total = x_ref.shape[1]
    nblk = t_total // blk
    npath = _NPATH if nblk % _NPATH == 0 and nblk >= _NPATH else 1
    per_path = nblk // npath
    for p in range(npath):
        lo, hi = p * per_path, (p + 1) * per_path
        cond = (i >= lo) & (i < hi) if p else (i < hi)

        @pl.when(cond)
        def _run(p=p):
            _path((p + 1) * per_path * blk)


def kernel(x, gain, bias, log_mix, log_scale):
    B, T, D = x.shape
    blk = min(256, T)
    grid = (B, T // blk)

    fn = functools.partial(_fused_kernel, blk=blk)
    return pl.pallas_call(
        fn,
        grid=grid,
        in_specs=[
            pl.BlockSpec((1, T, D), lambda b, i: (b, 0, 0)),
            pl.BlockSpec((1, D), lambda b, i: (0, 0)),
            pl.BlockSpec((1, D), lambda b, i: (0, 0)),
            pl.BlockSpec((1, 1), lambda b, i: (0, 0)),
            pl.BlockSpec((1, 1), lambda b, i: (0, 0)),
        ],
        out_specs=pl.BlockSpec((1, blk, D), lambda b, i: (b, i, 0)),
        out_shape=jax.ShapeDtypeStruct((B, T, D), x.dtype),
        scratch_shapes=[pltpu.VMEM((T, D), jnp.float32)],
    )(x, gain.reshape(1, D), bias.reshape(1, D),
      log_mix.reshape(1, 1), log_scale.reshape(1, 1))
